# baseline (speedup 1.0000x reference)
import jax
import jax.numpy as jnp
from jax import lax
from jax.experimental import pallas as pl
from jax.experimental.pallas import tpu as pltpu
from jax.experimental.pallas import tpu_sc as plsc

BATCH = 4096
HIST = 200
HPAD = 224
HALF = HPAD // 2
EMB = 64
NLANE = 16
NWORKERS = 32
ROWS = BATCH // NWORKERS
NSLOT = 8
GRP = ROWS // NLANE
NUM_ROWS = 1000000
PADROWS = 65536


def _body(x_hbm, m_hbm, pos_hbm, neg_hbm, tab_hbm, pos_out, neg_out,
          idx_c, xg_v, mskg_v, g_v, pidx_v, nidx_v, prow_v, nrow_v,
          ps_v, ns_v, cnt_v, nch_s, sems, sem_p, sem_n):
    wid = lax.axis_index("c") * 16 + lax.axis_index("s")
    base = wid * ROWS
    lanes = lax.iota(jnp.int32, NLANE)
    zero = jnp.zeros((NLANE,), jnp.float32)

    pltpu.sync_copy(pos_hbm.at[pl.ds(base, ROWS)], pidx_v)
    pltpu.sync_copy(neg_hbm.at[pl.ds(base, ROWS)], nidx_v)
    cp = pltpu.async_copy(tab_hbm.at[pidx_v], prow_v, sem_p)
    cn = pltpu.async_copy(tab_hbm.at[nidx_v], nrow_v, sem_n)

    def compact_group(gi, _):
        goff = base + gi * NLANE
        pltpu.sync_copy(x_hbm.at[pl.ds(goff, NLANE)], xg_v)
        pltpu.sync_copy(m_hbm.at[pl.ds(goff, NLANE)], mskg_v)
        cntv = zero
        for u in range(NLANE):
            r = gi * NLANE + u
            rbase = r * HPAD
            off = jnp.int32(0)
            for k in range(HPAD // NLANE):
                sl = pl.ds(k * NLANE, NLANE)
                pv = NUM_ROWS + ((rbase + k * NLANE + lanes) & (PADROWS - 1))
                idx_c[pl.ds(rbase + k * NLANE, NLANE)] = pv
                mv = mskg_v[u, sl]
                iv = xg_v[u, sl]
                pos = rbase + off + plsc.cumsum(mv) - 1
                plsc.store_scatter(idx_c, [pos], iv, mask=mv == 1)
                off = off + jnp.sum(mv)
            nch_s[r] = (off + NLANE - 1) // NLANE
            cntv = jnp.where(lanes == u, off.astype(jnp.float32), cntv)
        soff = pl.multiple_of(gi * NLANE, NLANE)
        cnt_v[pl.ds(soff, NLANE)] = cntv
        return 0

    lax.fori_loop(0, GRP, compact_group, 0)

    cp.wait()
    cn.wait()

    def issue_at(r, s):
        rbase = r * HPAD

        def q_issue(q, _):
            iv = idx_c[pl.ds(rbase + q * NLANE, NLANE)]
            pltpu.async_copy(tab_hbm.at[iv],
                             g_v.at[s, q], sems.at[s])
            return 0

        lax.fori_loop(0, nch_s[r], q_issue, 0)

    for s in range(NSLOT):
        issue_at(s, s)

    def do_row(r, carry):
        svp, svn = carry
        u = r % NLANE
        s0 = r % NSLOT
        nch = nch_s[r]

        def q_wait(q, _):
            pltpu.make_async_copy(tab_hbm.at[idx_c.at[pl.ds(0, NLANE)]],
                                  g_v.at[s0, 0], sems.at[s0]).wait()
            return 0

        lax.fori_loop(0, nch, q_wait, 0)

        def acc_step(l, accs):
            out = list(accs)
            q = l // NLANE
            i = l % NLANE
            for k in range(2):
                va = g_v[s0, q, i, pl.ds(k * 2 * NLANE, 2 * NLANE)]
                a, b = plsc.unpack(va, format=plsc.PackFormat.INTERLEAVED,
                                   preferred_element_type=jnp.float32)
                out[2 * k] = out[2 * k] + a
                out[2 * k + 1] = out[2 * k + 1] + b
            return tuple(out)

        accs = lax.fori_loop(0, nch * NLANE, acc_step, (zero,) * 4)

        @pl.when(r + NSLOT < ROWS)
        def _():
            issue_at(r + NSLOT, s0)

        acc = list(accs)
        pdot = zero
        ndot = zero
        for k in range(2):
            pa, pb = plsc.unpack(prow_v[r, pl.ds(k * 2 * NLANE, 2 * NLANE)],
                                 format=plsc.PackFormat.INTERLEAVED,
                                 preferred_element_type=jnp.float32)
            na, nb = plsc.unpack(nrow_v[r, pl.ds(k * 2 * NLANE, 2 * NLANE)],
                                 format=plsc.PackFormat.INTERLEAVED,
                                 preferred_element_type=jnp.float32)
            pdot = pdot + acc[2 * k] * pa + acc[2 * k + 1] * pb
            ndot = ndot + acc[2 * k] * na + acc[2 * k + 1] * nb
        svp = jnp.where(lanes == u, jnp.sum(pdot), svp)
        svn = jnp.where(lanes == u, jnp.sum(ndot), svn)

        @pl.when(u == NLANE - 1)
        def _():
            soff = pl.multiple_of(r - (NLANE - 1), NLANE)
            invv = 1.0 / jnp.maximum(cnt_v[pl.ds(soff, NLANE)], 1.0)
            ps_v[pl.ds(soff, NLANE)] = svp * invv
            ns_v[pl.ds(soff, NLANE)] = svn * invv

        return (svp, svn)

    lax.fori_loop(0, ROWS, do_row, (zero, zero))

    pltpu.sync_copy(ps_v, pos_out.at[pl.ds(base, ROWS)])
    pltpu.sync_copy(ns_v, neg_out.at[pl.ds(base, ROWS)])


@jax.jit
def _run(x2, m2, pos_items, neg_items, taby):
    mesh = plsc.VectorSubcoreMesh(core_axis_name="c", subcore_axis_name="s",
                                  num_cores=2, num_subcores=16)
    f = pl.kernel(
        _body,
        out_type=(
            jax.ShapeDtypeStruct((BATCH,), jnp.float32),
            jax.ShapeDtypeStruct((BATCH,), jnp.float32),
        ),
        mesh=mesh,
        compiler_params=pltpu.CompilerParams(needs_layout_passes=False,
                                             use_tc_tiling_on_sc=False),
        scratch_types=[
            pltpu.VMEM((ROWS * HPAD,), jnp.int32),
            pltpu.VMEM((NLANE, HPAD), jnp.int32),
            pltpu.VMEM((NLANE, HPAD), jnp.int32),
            pltpu.VMEM((NSLOT, HPAD // NLANE, NLANE, EMB),
                       jnp.bfloat16),
            pltpu.VMEM((ROWS,), jnp.int32),
            pltpu.VMEM((ROWS,), jnp.int32),
            pltpu.VMEM((ROWS, EMB), jnp.bfloat16),
            pltpu.VMEM((ROWS, EMB), jnp.bfloat16),
            pltpu.VMEM((ROWS,), jnp.float32),
            pltpu.VMEM((ROWS,), jnp.float32),
            pltpu.VMEM((ROWS,), jnp.float32),
            pltpu.SMEM((ROWS,), jnp.int32),
            pltpu.SemaphoreType.DMA((NSLOT,)),
            pltpu.SemaphoreType.DMA,
            pltpu.SemaphoreType.DMA,
        ],
    )
    return f(x2, m2, pos_items, neg_items, taby)


def kernel(x_pad, mask, pos_items, neg_items, item_emb):
    x = x_pad.astype(jnp.int32)
    m = mask.astype(jnp.int32)
    pad = HPAD - HIST
    x2 = jnp.pad(x, ((0, 0), (0, pad)))
    m2 = jnp.pad(m, ((0, 0), (0, pad)))
    taby = jnp.concatenate(
        [item_emb.astype(jnp.bfloat16),
         jnp.zeros((PADROWS, EMB), jnp.bfloat16)], axis=0)
    pos_score, neg_score = _run(x2, m2, pos_items.astype(jnp.int32),
                                neg_items.astype(jnp.int32), taby)
    return (pos_score, neg_score)


# speedup vs baseline: 1.0008x; 1.0008x over previous
import jax
import jax.numpy as jnp
from jax import lax
from jax.experimental import pallas as pl
from jax.experimental.pallas import tpu as pltpu
from jax.experimental.pallas import tpu_sc as plsc

BATCH = 4096
HIST = 200
HPAD = 224
HALF = HPAD // 2
EMB = 64
NLANE = 16
NWORKERS = 32
ROWS = BATCH // NWORKERS
NSLOT = 4
GRP = ROWS // NLANE
NUM_ROWS = 1000000
PADROWS = 65536


def _body(x_hbm, m_hbm, pos_hbm, neg_hbm, tab_hbm, pos_out, neg_out,
          idx_c, xg_v, mskg_v, g_v, pidx_v, nidx_v, prow_v, nrow_v,
          ps_v, ns_v, cnt_v, nch_s, sems, sem_p, sem_n):
    wid = lax.axis_index("c") * 16 + lax.axis_index("s")
    base = wid * ROWS
    lanes = lax.iota(jnp.int32, NLANE)
    zero = jnp.zeros((NLANE,), jnp.float32)

    pltpu.sync_copy(pos_hbm.at[pl.ds(base, ROWS)], pidx_v)
    pltpu.sync_copy(neg_hbm.at[pl.ds(base, ROWS)], nidx_v)
    cp = pltpu.async_copy(tab_hbm.at[pidx_v], prow_v, sem_p)
    cn = pltpu.async_copy(tab_hbm.at[nidx_v], nrow_v, sem_n)

    def compact_group(gi, _):
        goff = base + gi * NLANE
        pltpu.sync_copy(x_hbm.at[pl.ds(goff, NLANE)], xg_v)
        pltpu.sync_copy(m_hbm.at[pl.ds(goff, NLANE)], mskg_v)
        cntv = zero
        for u in range(NLANE):
            r = gi * NLANE + u
            rbase = r * HPAD
            off = jnp.int32(0)
            for k in range(HPAD // NLANE):
                sl = pl.ds(k * NLANE, NLANE)
                pv = NUM_ROWS + ((rbase + k * NLANE + lanes) & (PADROWS - 1))
                idx_c[pl.ds(rbase + k * NLANE, NLANE)] = pv
                mv = mskg_v[u, sl]
                iv = xg_v[u, sl]
                pos = rbase + off + plsc.cumsum(mv) - 1
                plsc.store_scatter(idx_c, [pos], iv, mask=mv == 1)
                off = off + jnp.sum(mv)
            nch_s[r] = (off + NLANE - 1) // NLANE
            cntv = jnp.where(lanes == u, off.astype(jnp.float32), cntv)
        soff = pl.multiple_of(gi * NLANE, NLANE)
        cnt_v[pl.ds(soff, NLANE)] = cntv
        return 0

    lax.fori_loop(0, GRP, compact_group, 0)

    cp.wait()
    cn.wait()

    def issue_at(r, s):
        rbase = r * HPAD

        def q_issue(q, _):
            iv = idx_c[pl.ds(rbase + q * NLANE, NLANE)]
            pltpu.async_copy(tab_hbm.at[iv],
                             g_v.at[s, q], sems.at[s])
            return 0

        lax.fori_loop(0, nch_s[r], q_issue, 0)

    for s in range(NSLOT):
        issue_at(s, s)

    def do_row(r, carry):
        svp, svn = carry
        u = r % NLANE
        s0 = r % NSLOT
        nch = nch_s[r]

        def q_wait(q, _):
            pltpu.make_async_copy(tab_hbm.at[idx_c.at[pl.ds(0, NLANE)]],
                                  g_v.at[s0, 0], sems.at[s0]).wait()
            return 0

        lax.fori_loop(0, nch, q_wait, 0)

        def acc_step(l, accs):
            out = list(accs)
            q = l // NLANE
            i = l % NLANE
            for k in range(2):
                va = g_v[s0, q, i, pl.ds(k * 2 * NLANE, 2 * NLANE)]
                a, b = plsc.unpack(va, format=plsc.PackFormat.INTERLEAVED,
                                   preferred_element_type=jnp.float32)
                out[2 * k] = out[2 * k] + a
                out[2 * k + 1] = out[2 * k + 1] + b
            return tuple(out)

        accs = lax.fori_loop(0, nch * NLANE, acc_step, (zero,) * 4)

        @pl.when(r + NSLOT < ROWS)
        def _():
            issue_at(r + NSLOT, s0)

        acc = list(accs)
        pdot = zero
        ndot = zero
        for k in range(2):
            pa, pb = plsc.unpack(prow_v[r, pl.ds(k * 2 * NLANE, 2 * NLANE)],
                                 format=plsc.PackFormat.INTERLEAVED,
                                 preferred_element_type=jnp.float32)
            na, nb = plsc.unpack(nrow_v[r, pl.ds(k * 2 * NLANE, 2 * NLANE)],
                                 format=plsc.PackFormat.INTERLEAVED,
                                 preferred_element_type=jnp.float32)
            pdot = pdot + acc[2 * k] * pa + acc[2 * k + 1] * pb
            ndot = ndot + acc[2 * k] * na + acc[2 * k + 1] * nb
        svp = jnp.where(lanes == u, jnp.sum(pdot), svp)
        svn = jnp.where(lanes == u, jnp.sum(ndot), svn)

        @pl.when(u == NLANE - 1)
        def _():
            soff = pl.multiple_of(r - (NLANE - 1), NLANE)
            invv = 1.0 / jnp.maximum(cnt_v[pl.ds(soff, NLANE)], 1.0)
            ps_v[pl.ds(soff, NLANE)] = svp * invv
            ns_v[pl.ds(soff, NLANE)] = svn * invv

        return (svp, svn)

    lax.fori_loop(0, ROWS, do_row, (zero, zero))

    pltpu.sync_copy(ps_v, pos_out.at[pl.ds(base, ROWS)])
    pltpu.sync_copy(ns_v, neg_out.at[pl.ds(base, ROWS)])


@jax.jit
def _run(x2, m2, pos_items, neg_items, taby):
    mesh = plsc.VectorSubcoreMesh(core_axis_name="c", subcore_axis_name="s",
                                  num_cores=2, num_subcores=16)
    f = pl.kernel(
        _body,
        out_type=(
            jax.ShapeDtypeStruct((BATCH,), jnp.float32),
            jax.ShapeDtypeStruct((BATCH,), jnp.float32),
        ),
        mesh=mesh,
        compiler_params=pltpu.CompilerParams(needs_layout_passes=False,
                                             use_tc_tiling_on_sc=False),
        scratch_types=[
            pltpu.VMEM((ROWS * HPAD,), jnp.int32),
            pltpu.VMEM((NLANE, HPAD), jnp.int32),
            pltpu.VMEM((NLANE, HPAD), jnp.int32),
            pltpu.VMEM((NSLOT, HPAD // NLANE, NLANE, EMB),
                       jnp.bfloat16),
            pltpu.VMEM((ROWS,), jnp.int32),
            pltpu.VMEM((ROWS,), jnp.int32),
            pltpu.VMEM((ROWS, EMB), jnp.bfloat16),
            pltpu.VMEM((ROWS, EMB), jnp.bfloat16),
            pltpu.VMEM((ROWS,), jnp.float32),
            pltpu.VMEM((ROWS,), jnp.float32),
            pltpu.VMEM((ROWS,), jnp.float32),
            pltpu.SMEM((ROWS,), jnp.int32),
            pltpu.SemaphoreType.DMA((NSLOT,)),
            pltpu.SemaphoreType.DMA,
            pltpu.SemaphoreType.DMA,
        ],
    )
    return f(x2, m2, pos_items, neg_items, taby)


def kernel(x_pad, mask, pos_items, neg_items, item_emb):
    x = x_pad.astype(jnp.int32)
    m = mask.astype(jnp.int32)
    pad = HPAD - HIST
    x2 = jnp.pad(x, ((0, 0), (0, pad)))
    m2 = jnp.pad(m, ((0, 0), (0, pad)))
    taby = jnp.concatenate(
        [item_emb.astype(jnp.bfloat16),
         jnp.zeros((PADROWS, EMB), jnp.bfloat16)], axis=0)
    pos_score, neg_score = _run(x2, m2, pos_items.astype(jnp.int32),
                                neg_items.astype(jnp.int32), taby)
    return (pos_score, neg_score)
